# baseline (speedup 1.0000x reference)
import functools

import jax
import jax.numpy as jnp
from jax.experimental import pallas as pl
from jax.experimental.pallas import tpu as pltpu

_VMEM_LIMIT = 65472 * 1024


def _gemm1_kernel(x_ref, w1_ref, b1_ref, h_ref):
    t = jnp.dot(x_ref[...], w1_ref[...], preferred_element_type=jnp.float32)
    h_ref[...] = jnp.maximum(t + b1_ref[...], 0.0).astype(jnp.bfloat16)


def _gemm2_kernel(h_ref, w2_ref, b2_ref, out_ref):
    t = jnp.dot(h_ref[...], w2_ref[...], preferred_element_type=jnp.float32)
    out_ref[...] = t + b2_ref[...]


@functools.partial(jax.jit, static_argnames=("blk_m1", "blk_ff", "blk_m2"))
def _ffn(x, W1, b1, W2, b2, blk_m1=512, blk_ff=2048, blk_m2=512):
    n_tok, d_model = x.shape
    d_ff = W1.shape[1]
    h = pl.pallas_call(
        _gemm1_kernel,
        grid=(d_ff // blk_ff, n_tok // blk_m1),
        in_specs=[
            pl.BlockSpec((blk_m1, d_model), lambda j, i: (i, 0)),
            pl.BlockSpec((d_model, blk_ff), lambda j, i: (0, j)),
            pl.BlockSpec((blk_ff,), lambda j, i: (j,)),
        ],
        out_specs=pl.BlockSpec((blk_m1, blk_ff), lambda j, i: (i, j)),
        out_shape=jax.ShapeDtypeStruct((n_tok, d_ff), jnp.bfloat16),
        compiler_params=pltpu.CompilerParams(
            dimension_semantics=("parallel", "arbitrary"),
            vmem_limit_bytes=_VMEM_LIMIT,
        ),
    )(x, W1, b1)
    blk_n = 1024
    return pl.pallas_call(
        _gemm2_kernel,
        grid=(d_model // blk_n, n_tok // blk_m2),
        in_specs=[
            pl.BlockSpec((blk_m2, d_ff), lambda j, i: (i, 0)),
            pl.BlockSpec((d_ff, blk_n), lambda j, i: (0, j)),
            pl.BlockSpec((blk_n,), lambda j, i: (j,)),
        ],
        out_specs=pl.BlockSpec((blk_m2, blk_n), lambda j, i: (i, j)),
        out_shape=jax.ShapeDtypeStruct((n_tok, d_model), jnp.float32),
        compiler_params=pltpu.CompilerParams(
            dimension_semantics=("arbitrary", "parallel"),
            vmem_limit_bytes=_VMEM_LIMIT,
        ),
    )(h, W2, b2)


def kernel(x, W1, b1, W2, b2):
    return _ffn(x, W1, b1, W2.astype(jnp.bfloat16), b2)
